# Initial kernel scaffold; baseline (speedup 1.0000x reference)
import jax
import jax.numpy as jnp
from jax.experimental import pallas as pl


def kernel(emb, mask_rows, mask_cols, rowsum):
    raise NotImplementedError("write your pallas kernel here")


# speedup vs baseline: 4.0225x; 4.0225x over previous
import functools
import jax
import jax.numpy as jnp
from jax import lax
from jax.experimental import pallas as pl
from jax.experimental.pallas import tpu as pltpu
from jax.experimental.pallas import tpu_sc as plsc

M = 16384
N = 16384
D = 256

NW = 32
RB = 128
NSB = M // RB
SBW = NSB // NW
K = 128
LG = 16
CG = D // LG


def _sc_segsum(emb, cols_i32, rows_i32, block_off):
  mesh = plsc.VectorSubcoreMesh(core_axis_name="c", subcore_axis_name="s")

  @functools.partial(
      pl.kernel,
      mesh=mesh,
      out_type=jax.ShapeDtypeStruct((M, D), jnp.float32),
      scratch_types=[
          pltpu.VMEM((K,), jnp.int32),
          pltpu.VMEM((K, D), jnp.float32),
          pltpu.VMEM((RB + 1, D), jnp.float32),
          pltpu.VMEM((NSB + 1 + LG,), jnp.int32),
          pltpu.VMEM((K,), jnp.int32),
          pltpu.SemaphoreType.DMA,
      ],
  )
  def segsum(emb_hbm, cols_hbm, rows_hbm, boff_hbm, out_hbm,
             idx_v, gath_v, acc_v, boff_v, rows_v, sem):
    wid = lax.axis_index("s") * 2 + lax.axis_index("c")
    pltpu.sync_copy(boff_hbm, boff_v)
    offs = boff_v[pl.ds(wid * SBW, LG)]

    zeros16 = jnp.zeros((LG,), jnp.float32)

    for k in range(SBW):
      rowbase = (wid * SBW + k) * RB
      e0 = offs[k]
      e1 = offs[k + 1]
      gstart = jnp.bitwise_and(e0, -8)
      nch = (e1 - gstart + (K - 1)) // K

      def zero_row(r, carry):
        for c in range(CG):
          acc_v[r, pl.ds(c * LG, LG)] = zeros16
        return carry

      lax.fori_loop(0, RB + 1, zero_row, 0)

      def do_chunk(ci, carry):
        g = pl.multiple_of(gstart + ci * K, 8)
        pltpu.sync_copy(cols_hbm.at[pl.ds(g, K)], idx_v)
        pltpu.async_copy(emb_hbm.at[idx_v], gath_v, sem).wait()
        pltpu.sync_copy(rows_hbm.at[pl.ds(g, K)], rows_v)

        def do_group(gi, c2):
          lr16 = rows_v[pl.ds(gi * LG, LG)] - rowbase
          bad = jnp.logical_or(lr16 < 0, lr16 >= RB)
          lr16 = jnp.where(bad, RB, lr16)
          for lane in range(LG):
            lr = lr16[lane]
            e = gi * LG + lane
            for c in range(CG):
              plsc.addupdate(acc_v.at[lr, pl.ds(c * LG, LG)],
                             gath_v[e, pl.ds(c * LG, LG)])
          return c2

        lax.fori_loop(0, K // LG, do_group, 0)
        return carry

      lax.fori_loop(0, nch, do_chunk, 0)

      pltpu.sync_copy(acc_v.at[pl.ds(0, RB)], out_hbm.at[pl.ds(rowbase, RB)])

  return segsum(emb, cols_i32, rows_i32, block_off)


_TC_BR = 1024


def _tc_norm_body(vsum_ref, rowsum_ref, o_ref):
  v = vsum_ref[...] / rowsum_ref[...]
  n = jnp.sqrt(jnp.sum(v * v, axis=1, keepdims=True))
  o_ref[...] = v / jnp.maximum(n, 1e-12)


def _tc_normalize(vsum, rowsum):
  grid = (M // _TC_BR,)
  return pl.pallas_call(
      _tc_norm_body,
      grid=grid,
      in_specs=[
          pl.BlockSpec((_TC_BR, D), lambda i: (i, 0)),
          pl.BlockSpec((_TC_BR, 1), lambda i: (i, 0)),
      ],
      out_specs=pl.BlockSpec((_TC_BR, D), lambda i: (i, 0)),
      out_shape=jax.ShapeDtypeStruct((M, D), jnp.float32),
  )(vsum, rowsum)


def kernel(emb, mask_rows, mask_cols, rowsum):
  rows_i32 = mask_rows.astype(jnp.int32)
  cols_i32 = mask_cols.astype(jnp.int32)

  pad = 2 * K
  rows_i32 = jnp.concatenate(
      [rows_i32, jnp.full((pad,), M, dtype=jnp.int32)])
  cols_i32 = jnp.concatenate(
      [cols_i32, jnp.zeros((pad,), dtype=jnp.int32)])

  block_off = jnp.searchsorted(
      mask_rows, jnp.arange(0, M + 1, RB, dtype=mask_rows.dtype),
      side="left").astype(jnp.int32)
  block_off = jnp.concatenate(
      [block_off, jnp.zeros((LG,), dtype=jnp.int32)])

  vsum = _sc_segsum(emb, cols_i32, rows_i32, block_off)
  return _tc_normalize(vsum, rowsum)
